# Initial kernel scaffold; baseline (speedup 1.0000x reference)
import jax
import jax.numpy as jnp
from jax.experimental import pallas as pl


def kernel(X, table):
    raise NotImplementedError("write your pallas kernel here")


# speedup vs baseline: 4.5387x; 4.5387x over previous
import functools

import jax
import jax.numpy as jnp
from jax import lax
from jax.experimental import pallas as pl
from jax.experimental.pallas import tpu as pltpu
from jax.experimental.pallas import tpu_sc as plsc

VOCAB = 100000
EMBED_DIM = 64
BATCH = 4096
HIST_LEN = 50

L = 128
B = BATCH * HIST_LEN
R = B // L
NC, NS = 2, 16
NW = NC * NS
ROWS_PER_W = R // NW
GROUP = 5
NGROUP = ROWS_PER_W // GROUP

_mesh = plsc.VectorSubcoreMesh(core_axis_name="c", subcore_axis_name="s")


@functools.partial(
    pl.kernel,
    out_type=jax.ShapeDtypeStruct((NW, ROWS_PER_W, L, EMBED_DIM), jnp.float32),
    mesh=_mesh,
    scratch_types=[
        pltpu.VMEM((ROWS_PER_W, L), jnp.int32),
        pltpu.VMEM((GROUP, L, EMBED_DIM), jnp.float32),
        pltpu.SemaphoreType.DMA,
    ],
    compiler_params=pltpu.CompilerParams(use_tc_tiling_on_sc=False),
)
def _sc_gather(table_hbm, idx_hbm, out_hbm, idx_v, rows_v, gsem):
    wid = lax.axis_index("s") * NC + lax.axis_index("c")
    pltpu.sync_copy(idx_hbm.at[wid], idx_v)
    for g in range(NGROUP):
        waits = []
        for j in range(GROUP):
            waits.append(pltpu.async_copy(
                table_hbm.at[idx_v.at[g * GROUP + j]], rows_v.at[j], gsem))
        for w in waits:
            w.wait()
        pltpu.sync_copy(rows_v, out_hbm.at[wid, pl.ds(g * GROUP, GROUP)])


def kernel(X, table):
    idx = X.reshape(NW, ROWS_PER_W, L).astype(jnp.int32)
    out = _sc_gather(table, idx)
    return out.reshape(BATCH, HIST_LEN, EMBED_DIM)


# speedup vs baseline: 4.6074x; 1.0151x over previous
import functools

import jax
import jax.numpy as jnp
from jax import lax
from jax.experimental import pallas as pl
from jax.experimental.pallas import tpu as pltpu
from jax.experimental.pallas import tpu_sc as plsc

VOCAB = 100000
EMBED_DIM = 64
BATCH = 4096
HIST_LEN = 50

L = 128
B = BATCH * HIST_LEN
R = B // L
NC, NS = 2, 16
NW = NC * NS
ROWS_PER_W = R // NW
GROUP = 5
NGROUP = ROWS_PER_W // GROUP

_mesh = plsc.VectorSubcoreMesh(core_axis_name="c", subcore_axis_name="s")


@functools.partial(
    pl.kernel,
    out_type=jax.ShapeDtypeStruct((NW, ROWS_PER_W, L, EMBED_DIM), jnp.float32),
    mesh=_mesh,
    scratch_types=[
        pltpu.VMEM((ROWS_PER_W, L), jnp.int32),
        pltpu.VMEM((2, GROUP, L, EMBED_DIM), jnp.float32),
        pltpu.SemaphoreType.DMA,
        pltpu.SemaphoreType.DMA,
        pltpu.SemaphoreType.DMA,
    ],
    compiler_params=pltpu.CompilerParams(use_tc_tiling_on_sc=False),
)
def _sc_gather(table_hbm, idx_hbm, out_hbm, idx_v, rows_v, gsem, osem0, osem1):
    wid = lax.axis_index("s") * NC + lax.axis_index("c")
    osems = (osem0, osem1)
    pltpu.sync_copy(idx_hbm.at[wid], idx_v)
    out_waits = [None, None]
    for g in range(NGROUP):
        buf = g % 2
        if out_waits[buf] is not None:
            out_waits[buf].wait()
        gather_waits = []
        for j in range(GROUP):
            gather_waits.append(pltpu.async_copy(
                table_hbm.at[idx_v.at[g * GROUP + j]], rows_v.at[buf, j],
                gsem))
        for w in gather_waits:
            w.wait()
        out_waits[buf] = pltpu.async_copy(
            rows_v.at[buf], out_hbm.at[wid, pl.ds(g * GROUP, GROUP)],
            osems[buf])
    out_waits[0].wait()
    out_waits[1].wait()


def kernel(X, table):
    idx = X.reshape(NW, ROWS_PER_W, L).astype(jnp.int32)
    out = _sc_gather(table, idx)
    return out.reshape(BATCH, HIST_LEN, EMBED_DIM)


# speedup vs baseline: 4.6460x; 1.0084x over previous
import functools

import jax
import jax.numpy as jnp
from jax import lax
from jax.experimental import pallas as pl
from jax.experimental.pallas import tpu as pltpu
from jax.experimental.pallas import tpu_sc as plsc

VOCAB = 100000
EMBED_DIM = 64
BATCH = 4096
HIST_LEN = 50

L = 128
B = BATCH * HIST_LEN
R = B // L
NC, NS = 2, 16
NW = NC * NS
ROWS_PER_W = R // NW
GROUP = 5
NGROUP = ROWS_PER_W // GROUP

_mesh = plsc.VectorSubcoreMesh(core_axis_name="c", subcore_axis_name="s")


@functools.partial(
    pl.kernel,
    out_type=jax.ShapeDtypeStruct((NW, ROWS_PER_W, L, EMBED_DIM), jnp.float32),
    mesh=_mesh,
    scratch_types=[
        pltpu.VMEM((ROWS_PER_W, L), jnp.int32),
        pltpu.VMEM((2, GROUP, L, EMBED_DIM), jnp.float32),
        pltpu.SemaphoreType.DMA,
        pltpu.SemaphoreType.DMA,
        pltpu.SemaphoreType.DMA,
        pltpu.SemaphoreType.DMA,
    ],
    compiler_params=pltpu.CompilerParams(use_tc_tiling_on_sc=False),
)
def _sc_gather(table_hbm, idx_hbm, out_hbm, idx_v, rows_v,
               gsem0, gsem1, osem0, osem1):
    wid = lax.axis_index("s") * NC + lax.axis_index("c")
    gsems, osems = (gsem0, gsem1), (osem0, osem1)
    pltpu.sync_copy(idx_hbm.at[wid], idx_v)
    gather_waits = [None, None]
    out_waits = [None, None]

    def fire(g):
        buf = g % 2
        gather_waits[buf] = [
            pltpu.async_copy(
                table_hbm.at[idx_v.at[g * GROUP + j]], rows_v.at[buf, j],
                gsems[buf])
            for j in range(GROUP)
        ]

    def drain(g):
        buf = g % 2
        for w in gather_waits[buf]:
            w.wait()
        out_waits[buf] = pltpu.async_copy(
            rows_v.at[buf], out_hbm.at[wid, pl.ds(g * GROUP, GROUP)],
            osems[buf])

    fire(0)
    for g in range(1, NGROUP):
        buf = g % 2
        if out_waits[buf] is not None:
            out_waits[buf].wait()
        fire(g)
        drain(g - 1)
    drain(NGROUP - 1)
    out_waits[0].wait()
    out_waits[1].wait()


def kernel(X, table):
    idx = X.reshape(NW, ROWS_PER_W, L).astype(jnp.int32)
    out = _sc_gather(table, idx)
    return out.reshape(BATCH, HIST_LEN, EMBED_DIM)
